# Initial kernel scaffold; baseline (speedup 1.0000x reference)
import jax
import jax.numpy as jnp
from jax.experimental import pallas as pl


def kernel(prem_ids, hyp_ids, table, W1, b1, W2, b2, add_noise):
    raise NotImplementedError("write your pallas kernel here")


# speedup vs baseline: 14.8360x; 14.8360x over previous
import functools

import jax
import jax.numpy as jnp
from jax import lax
from jax.experimental import pallas as pl
from jax.experimental.pallas import tpu as pltpu
from jax.experimental.pallas import tpu_sc as plsc

B, L, V, D = 4096, 50, 100000, 128
NC, NS = 2, 16
NW = NC * NS
SEG_PER_SIDE = B // NW
NBUF = 4
NCOL = D // 16
INV_L = 1.0 / L

_MESH = plsc.VectorSubcoreMesh(
    core_axis_name="c", subcore_axis_name="s", num_cores=NC, num_subcores=NS)


@functools.partial(
    pl.kernel,
    out_type=(jax.ShapeDtypeStruct((B, D), jnp.float32),
              jax.ShapeDtypeStruct((B, D), jnp.float32)),
    mesh=_MESH,
    scratch_types=[
        pltpu.VMEM((SEG_PER_SIDE, L), jnp.int32),
        pltpu.VMEM((NBUF, L, D), jnp.float32),
        pltpu.VMEM((SEG_PER_SIDE, D), jnp.float32),
        pltpu.SemaphoreType.DMA((NBUF,)),
    ],
)
def _sc_embed_mean(prem_hbm, hyp_hbm, table_hbm, vp_hbm, vh_hbm,
                   idx_v, rows_v, out_v, sems):
    wid = lax.axis_index("s") * NC + lax.axis_index("c")
    base = wid * SEG_PER_SIDE

    def one_side(ids_hbm, out_hbm):
        pltpu.sync_copy(ids_hbm.at[pl.ds(base, SEG_PER_SIDE)], idx_v)
        for b in range(NBUF):
            pltpu.async_copy(table_hbm.at[idx_v.at[b]], rows_v.at[b],
                             sems.at[b])

        def outer(g2, carry):
            for b in range(NBUF):
                g = g2 * NBUF + b
                pltpu.make_async_copy(table_hbm.at[idx_v.at[0]],
                                      rows_v.at[b], sems.at[b]).wait()

                def acc_body(l, accs):
                    return tuple(accs[c] + rows_v[b, l, pl.ds(16 * c, 16)]
                                 for c in range(NCOL))

                accs = lax.fori_loop(
                    0, L, acc_body,
                    tuple(jnp.zeros((16,), jnp.float32)
                          for _ in range(NCOL)),
                    unroll=5)
                for c in range(NCOL):
                    out_v[g, pl.ds(16 * c, 16)] = accs[c] * INV_L

                nxt = g + NBUF

                @pl.when(nxt < SEG_PER_SIDE)
                def _():
                    pltpu.async_copy(table_hbm.at[idx_v.at[nxt]],
                                     rows_v.at[b], sems.at[b])
            return carry

        lax.fori_loop(0, SEG_PER_SIDE // NBUF, outer, 0)
        pltpu.sync_copy(out_v, out_hbm.at[pl.ds(base, SEG_PER_SIDE)])

    one_side(prem_hbm, vp_hbm)
    one_side(hyp_hbm, vh_hbm)


_SQRT_HALF = 0.7071067811865476
_BM = 512


def _tc_mlp_body(vp_ref, vh_ref, w1_ref, b1_ref, w2_ref, b2_ref, h0_ref):
    w1 = w1_ref[...]
    b1 = b1_ref[...]
    w2 = w2_ref[...]
    b2 = b2_ref[...]

    def mlp(v):
        h = jnp.dot(v, w1, preferred_element_type=jnp.float32) + b1
        h = 0.5 * h * (1.0 + lax.erf(h * _SQRT_HALF))
        return jnp.dot(h, w2, preferred_element_type=jnp.float32)

    h0_ref[...] = mlp(vp_ref[...]) + mlp(vh_ref[...]) + 2.0 * b2


def _tc_mlp(v_p, v_h, W1, b1, W2, b2):
    return pl.pallas_call(
        _tc_mlp_body,
        grid=(B // _BM,),
        in_specs=[
            pl.BlockSpec((_BM, D), lambda i: (i, 0)),
            pl.BlockSpec((_BM, D), lambda i: (i, 0)),
            pl.BlockSpec((D, 2 * D), lambda i: (0, 0)),
            pl.BlockSpec((1, 2 * D), lambda i: (0, 0)),
            pl.BlockSpec((2 * D, D), lambda i: (0, 0)),
            pl.BlockSpec((1, D), lambda i: (0, 0)),
        ],
        out_specs=pl.BlockSpec((_BM, D), lambda i: (i, 0)),
        out_shape=jax.ShapeDtypeStruct((B, D), jnp.float32),
    )(v_p, v_h, W1, b1.reshape(1, -1), W2, b2.reshape(1, -1))


def kernel(prem_ids, hyp_ids, table, W1, b1, W2, b2, add_noise=0):
    del add_noise
    prem = prem_ids.astype(jnp.int32)
    hyp = hyp_ids.astype(jnp.int32)
    v_p, v_h = _sc_embed_mean(prem, hyp, table)
    h0 = _tc_mlp(v_p, v_h, W1, b1, W2, b2)
    return (h0, v_p, v_h)


# speedup vs baseline: 17.3179x; 1.1673x over previous
import functools

import jax
import jax.numpy as jnp
from jax import lax
from jax.experimental import pallas as pl
from jax.experimental.pallas import tpu as pltpu
from jax.experimental.pallas import tpu_sc as plsc

B, L, V, D = 4096, 50, 100000, 128
NC, NS = 2, 16
NW = NC * NS
SEG_PER_SIDE = B // NW
PAIR_PER_SIDE = SEG_PER_SIDE // 2
L2 = 2 * L
NBUF = 4
NCOL = D // 16
INV_L = 1.0 / L

_MESH = plsc.VectorSubcoreMesh(
    core_axis_name="c", subcore_axis_name="s", num_cores=NC, num_subcores=NS)


@functools.partial(
    pl.kernel,
    out_type=(jax.ShapeDtypeStruct((B, D), jnp.float32),
              jax.ShapeDtypeStruct((B, D), jnp.float32)),
    mesh=_MESH,
    scratch_types=[
        pltpu.VMEM((PAIR_PER_SIDE, L2), jnp.int32),
        pltpu.VMEM((NBUF, L2, D), jnp.float32),
        pltpu.VMEM((SEG_PER_SIDE, D), jnp.float32),
        pltpu.SemaphoreType.DMA((NBUF,)),
    ],
)
def _sc_embed_mean(ids_hbm, table_hbm, vp_hbm, vh_hbm,
                   idx_v, rows_v, out_v, sems):
    wid = lax.axis_index("s") * NC + lax.axis_index("c")
    base = wid * SEG_PER_SIDE
    pair_base = wid * PAIR_PER_SIDE

    def one_side(row_off, out_hbm):
        pltpu.sync_copy(
            ids_hbm.at[pl.ds(row_off + pair_base, PAIR_PER_SIDE)], idx_v)
        for b in range(NBUF):
            pltpu.async_copy(table_hbm.at[idx_v.at[b]], rows_v.at[b],
                             sems.at[b])

        def outer(g2, carry):
            for b in range(NBUF):
                g = g2 * NBUF + b
                pltpu.make_async_copy(table_hbm.at[idx_v.at[0]],
                                      rows_v.at[b], sems.at[b]).wait()

                zeros = tuple(jnp.zeros((16,), jnp.float32)
                              for _ in range(NCOL))

                def acc_body(l, accs):
                    a, bb = accs
                    a = tuple(a[c] + rows_v[b, l, pl.ds(16 * c, 16)]
                              for c in range(NCOL))
                    bb = tuple(bb[c] + rows_v[b, l + L, pl.ds(16 * c, 16)]
                               for c in range(NCOL))
                    return (a, bb)

                acc_a, acc_b = plsc.parallel_loop(
                    0, L, unroll=4, carry=(zeros, zeros))(acc_body)
                for c in range(NCOL):
                    out_v[2 * g, pl.ds(16 * c, 16)] = acc_a[c] * INV_L
                    out_v[2 * g + 1, pl.ds(16 * c, 16)] = acc_b[c] * INV_L

                nxt = g + NBUF

                @pl.when(nxt < PAIR_PER_SIDE)
                def _():
                    pltpu.async_copy(table_hbm.at[idx_v.at[nxt]],
                                     rows_v.at[b], sems.at[b])
            return carry

        lax.fori_loop(0, PAIR_PER_SIDE // NBUF, outer, 0)
        pltpu.sync_copy(out_v, out_hbm.at[pl.ds(base, SEG_PER_SIDE)])

    one_side(0, vp_hbm)
    one_side(B // 2, vh_hbm)


_SQRT_HALF = 0.7071067811865476
_BM = 1024


def _tc_mlp_body(vp_ref, vh_ref, w1_ref, b1_ref, w2_ref, b2_ref, h0_ref):
    w1 = w1_ref[...]
    b1 = b1_ref[...]
    w2 = w2_ref[...]
    b2 = b2_ref[...]

    def mlp(v):
        h = jnp.dot(v, w1, preferred_element_type=jnp.float32) + b1
        h = 0.5 * h * (1.0 + lax.erf(h * _SQRT_HALF))
        return jnp.dot(h, w2, preferred_element_type=jnp.float32)

    h0_ref[...] = mlp(vp_ref[...]) + mlp(vh_ref[...]) + 2.0 * b2


def _tc_mlp(v_p, v_h, W1, b1, W2, b2):
    return pl.pallas_call(
        _tc_mlp_body,
        grid=(B // _BM,),
        in_specs=[
            pl.BlockSpec((_BM, D), lambda i: (i, 0)),
            pl.BlockSpec((_BM, D), lambda i: (i, 0)),
            pl.BlockSpec((D, 2 * D), lambda i: (0, 0)),
            pl.BlockSpec((1, 2 * D), lambda i: (0, 0)),
            pl.BlockSpec((2 * D, D), lambda i: (0, 0)),
            pl.BlockSpec((1, D), lambda i: (0, 0)),
        ],
        out_specs=pl.BlockSpec((_BM, D), lambda i: (i, 0)),
        out_shape=jax.ShapeDtypeStruct((B, D), jnp.float32),
    )(v_p, v_h, W1, b1.reshape(1, -1), W2, b2.reshape(1, -1))


def kernel(prem_ids, hyp_ids, table, W1, b1, W2, b2, add_noise=0):
    del add_noise
    ids = jnp.concatenate(
        [prem_ids.astype(jnp.int32).reshape(B // 2, L2),
         hyp_ids.astype(jnp.int32).reshape(B // 2, L2)], axis=0)
    v_p, v_h = _sc_embed_mean(ids, table)
    h0 = _tc_mlp(v_p, v_h, W1, b1, W2, b2)
    return (h0, v_p, v_h)


# speedup vs baseline: 17.8544x; 1.0310x over previous
import functools

import jax
import jax.numpy as jnp
from jax import lax
from jax.experimental import pallas as pl
from jax.experimental.pallas import tpu as pltpu
from jax.experimental.pallas import tpu_sc as plsc

B, L, V, D = 4096, 50, 100000, 128
NC, NS = 2, 16
NW = NC * NS
SEG_PER_SIDE = B // NW
PAIR_PER_SIDE = SEG_PER_SIDE // 2
L2 = 2 * L
NBUF = 4
NCOL = D // 16
INV_L = 1.0 / L

_MESH = plsc.VectorSubcoreMesh(
    core_axis_name="c", subcore_axis_name="s", num_cores=NC, num_subcores=NS)


@functools.partial(
    pl.kernel,
    out_type=(jax.ShapeDtypeStruct((B, D), jnp.float32),
              jax.ShapeDtypeStruct((B, D), jnp.float32)),
    mesh=_MESH,
    scratch_types=[
        pltpu.VMEM((2 * PAIR_PER_SIDE, L2), jnp.int32),
        pltpu.VMEM((NBUF, L2, D), jnp.float32),
        pltpu.VMEM((2 * SEG_PER_SIDE, D), jnp.float32),
        pltpu.SemaphoreType.DMA((NBUF,)),
    ],
)
def _sc_embed_mean(ids_hbm, table_hbm, vp_hbm, vh_hbm,
                   idx_v, rows_v, out_v, sems):
    wid = lax.axis_index("s") * NC + lax.axis_index("c")
    base = wid * SEG_PER_SIDE
    pair_base = wid * PAIR_PER_SIDE
    npair = 2 * PAIR_PER_SIDE

    pltpu.sync_copy(ids_hbm.at[pl.ds(pair_base, PAIR_PER_SIDE)],
                    idx_v.at[pl.ds(0, PAIR_PER_SIDE)])
    pltpu.sync_copy(ids_hbm.at[pl.ds(B // 2 + pair_base, PAIR_PER_SIDE)],
                    idx_v.at[pl.ds(PAIR_PER_SIDE, PAIR_PER_SIDE)])
    for b in range(NBUF):
        pltpu.async_copy(table_hbm.at[idx_v.at[b]], rows_v.at[b],
                         sems.at[b])

    def outer(g2, carry):
        for b in range(NBUF):
            g = g2 * NBUF + b
            pltpu.make_async_copy(table_hbm.at[idx_v.at[0]],
                                  rows_v.at[b], sems.at[b]).wait()

            zeros = tuple(jnp.zeros((16,), jnp.float32)
                          for _ in range(NCOL))

            def acc_body(l, accs):
                a, bb = accs
                a = tuple(a[c] + rows_v[b, l, pl.ds(16 * c, 16)]
                          for c in range(NCOL))
                bb = tuple(bb[c] + rows_v[b, l + L, pl.ds(16 * c, 16)]
                           for c in range(NCOL))
                return (a, bb)

            acc_a, acc_b = plsc.parallel_loop(
                0, L, unroll=4, carry=(zeros, zeros))(acc_body)
            for c in range(NCOL):
                out_v[2 * g, pl.ds(16 * c, 16)] = acc_a[c] * INV_L
                out_v[2 * g + 1, pl.ds(16 * c, 16)] = acc_b[c] * INV_L

            nxt = g + NBUF

            @pl.when(nxt < npair)
            def _():
                pltpu.async_copy(table_hbm.at[idx_v.at[nxt]],
                                 rows_v.at[b], sems.at[b])
        return carry

    lax.fori_loop(0, npair // NBUF, outer, 0)
    pltpu.sync_copy(out_v.at[pl.ds(0, SEG_PER_SIDE)],
                    vp_hbm.at[pl.ds(base, SEG_PER_SIDE)])
    pltpu.sync_copy(out_v.at[pl.ds(SEG_PER_SIDE, SEG_PER_SIDE)],
                    vh_hbm.at[pl.ds(base, SEG_PER_SIDE)])


_SQRT_HALF = 0.7071067811865476
_BM = 1024


def _tc_mlp_body(vp_ref, vh_ref, w1_ref, b1_ref, w2_ref, b2_ref, h0_ref):
    w1 = w1_ref[...]
    b1 = b1_ref[...]
    w2 = w2_ref[...]
    b2 = b2_ref[...]

    def mlp(v):
        h = jnp.dot(v, w1, preferred_element_type=jnp.float32) + b1
        h = 0.5 * h * (1.0 + lax.erf(h * _SQRT_HALF))
        return jnp.dot(h, w2, preferred_element_type=jnp.float32)

    h0_ref[...] = mlp(vp_ref[...]) + mlp(vh_ref[...]) + 2.0 * b2


def _tc_mlp(v_p, v_h, W1, b1, W2, b2):
    return pl.pallas_call(
        _tc_mlp_body,
        grid=(B // _BM,),
        in_specs=[
            pl.BlockSpec((_BM, D), lambda i: (i, 0)),
            pl.BlockSpec((_BM, D), lambda i: (i, 0)),
            pl.BlockSpec((D, 2 * D), lambda i: (0, 0)),
            pl.BlockSpec((1, 2 * D), lambda i: (0, 0)),
            pl.BlockSpec((2 * D, D), lambda i: (0, 0)),
            pl.BlockSpec((1, D), lambda i: (0, 0)),
        ],
        out_specs=pl.BlockSpec((_BM, D), lambda i: (i, 0)),
        out_shape=jax.ShapeDtypeStruct((B, D), jnp.float32),
    )(v_p, v_h, W1, b1.reshape(1, -1), W2, b2.reshape(1, -1))


def kernel(prem_ids, hyp_ids, table, W1, b1, W2, b2, add_noise=0):
    del add_noise
    ids = jnp.concatenate(
        [prem_ids.astype(jnp.int32).reshape(B // 2, L2),
         hyp_ids.astype(jnp.int32).reshape(B // 2, L2)], axis=0)
    v_p, v_h = _sc_embed_mean(ids, table)
    h0 = _tc_mlp(v_p, v_h, W1, b1, W2, b2)
    return (h0, v_p, v_h)


# speedup vs baseline: 17.9304x; 1.0043x over previous
import functools

import jax
import jax.numpy as jnp
from jax import lax
from jax.experimental import pallas as pl
from jax.experimental.pallas import tpu as pltpu
from jax.experimental.pallas import tpu_sc as plsc

B, L, V, D = 4096, 50, 100000, 128
NC, NS = 2, 16
NW = NC * NS
SEG_PER_SIDE = B // NW
PAIR_PER_SIDE = SEG_PER_SIDE // 2
L2 = 2 * L
NBUF = 4
NCOL = D // 16
INV_L = 1.0 / L

_MESH = plsc.VectorSubcoreMesh(
    core_axis_name="c", subcore_axis_name="s", num_cores=NC, num_subcores=NS)


@functools.partial(
    pl.kernel,
    out_type=(jax.ShapeDtypeStruct((B, D), jnp.float32),
              jax.ShapeDtypeStruct((B, D), jnp.float32)),
    mesh=_MESH,
    scratch_types=[
        pltpu.VMEM((2 * PAIR_PER_SIDE, L2), jnp.int32),
        pltpu.VMEM((NBUF, L2, D), jnp.float32),
        pltpu.VMEM((2 * SEG_PER_SIDE, D), jnp.float32),
        pltpu.SemaphoreType.DMA((NBUF,)),
        pltpu.SemaphoreType.DMA,
    ],
)
def _sc_embed_mean(ids_hbm, table_hbm, vp_hbm, vh_hbm,
                   idx_v, rows_v, out_v, sems, out_sem):
    wid = lax.axis_index("s") * NC + lax.axis_index("c")
    base = wid * SEG_PER_SIDE
    pair_base = wid * PAIR_PER_SIDE
    npair = 2 * PAIR_PER_SIDE

    pltpu.sync_copy(ids_hbm.at[pl.ds(pair_base, PAIR_PER_SIDE)],
                    idx_v.at[pl.ds(0, PAIR_PER_SIDE)])
    pltpu.sync_copy(ids_hbm.at[pl.ds(B // 2 + pair_base, PAIR_PER_SIDE)],
                    idx_v.at[pl.ds(PAIR_PER_SIDE, PAIR_PER_SIDE)])
    for b in range(NBUF):
        pltpu.async_copy(table_hbm.at[idx_v.at[b]], rows_v.at[b],
                         sems.at[b])

    def outer(g2, carry):
        for b in range(NBUF):
            g = g2 * NBUF + b
            pltpu.make_async_copy(table_hbm.at[idx_v.at[0]],
                                  rows_v.at[b], sems.at[b]).wait()

            zeros = tuple(jnp.zeros((16,), jnp.float32)
                          for _ in range(NCOL))

            def acc_body(l, accs):
                a, bb = accs
                a = tuple(a[c] + rows_v[b, l, pl.ds(16 * c, 16)]
                          for c in range(NCOL))
                bb = tuple(bb[c] + rows_v[b, l + L, pl.ds(16 * c, 16)]
                           for c in range(NCOL))
                return (a, bb)

            acc_a, acc_b = plsc.parallel_loop(
                0, L, unroll=4, carry=(zeros, zeros))(acc_body)
            for c in range(NCOL):
                out_v[2 * g, pl.ds(16 * c, 16)] = acc_a[c] * INV_L
                out_v[2 * g + 1, pl.ds(16 * c, 16)] = acc_b[c] * INV_L

            nxt = g + NBUF

            @pl.when(nxt < npair)
            def _():
                pltpu.async_copy(table_hbm.at[idx_v.at[nxt]],
                                 rows_v.at[b], sems.at[b])

        @pl.when(g2 == PAIR_PER_SIDE // NBUF - 1)
        def _():
            pltpu.async_copy(out_v.at[pl.ds(0, SEG_PER_SIDE)],
                             vp_hbm.at[pl.ds(base, SEG_PER_SIDE)], out_sem)
        return carry

    lax.fori_loop(0, npair // NBUF, outer, 0)
    pltpu.sync_copy(out_v.at[pl.ds(SEG_PER_SIDE, SEG_PER_SIDE)],
                    vh_hbm.at[pl.ds(base, SEG_PER_SIDE)])
    pltpu.make_async_copy(out_v.at[pl.ds(0, SEG_PER_SIDE)],
                          vp_hbm.at[pl.ds(base, SEG_PER_SIDE)],
                          out_sem).wait()


_SQRT_HALF = 0.7071067811865476
_BM = 2048


def _tc_mlp_body(vp_ref, vh_ref, w1_ref, b1_ref, w2_ref, b2_ref, h0_ref):
    w1 = w1_ref[...]
    b1 = b1_ref[...]
    w2 = w2_ref[...]
    b2 = b2_ref[...]

    def mlp(v):
        h = jnp.dot(v, w1, preferred_element_type=jnp.float32) + b1
        h = 0.5 * h * (1.0 + lax.erf(h * _SQRT_HALF))
        return jnp.dot(h, w2, preferred_element_type=jnp.float32)

    h0_ref[...] = mlp(vp_ref[...]) + mlp(vh_ref[...]) + 2.0 * b2


def _tc_mlp(v_p, v_h, W1, b1, W2, b2):
    return pl.pallas_call(
        _tc_mlp_body,
        grid=(B // _BM,),
        in_specs=[
            pl.BlockSpec((_BM, D), lambda i: (i, 0)),
            pl.BlockSpec((_BM, D), lambda i: (i, 0)),
            pl.BlockSpec((D, 2 * D), lambda i: (0, 0)),
            pl.BlockSpec((1, 2 * D), lambda i: (0, 0)),
            pl.BlockSpec((2 * D, D), lambda i: (0, 0)),
            pl.BlockSpec((1, D), lambda i: (0, 0)),
        ],
        out_specs=pl.BlockSpec((_BM, D), lambda i: (i, 0)),
        out_shape=jax.ShapeDtypeStruct((B, D), jnp.float32),
    )(v_p, v_h, W1, b1.reshape(1, -1), W2, b2.reshape(1, -1))


def kernel(prem_ids, hyp_ids, table, W1, b1, W2, b2, add_noise=0):
    del add_noise
    ids = jnp.concatenate(
        [prem_ids.astype(jnp.int32).reshape(B // 2, L2),
         hyp_ids.astype(jnp.int32).reshape(B // 2, L2)], axis=0)
    v_p, v_h = _sc_embed_mean(ids, table)
    h0 = _tc_mlp(v_p, v_h, W1, b1, W2, b2)
    return (h0, v_p, v_h)
